# baseline (speedup 1.0000x reference)
import jax
import jax.numpy as jnp
from jax import lax
from jax.experimental import pallas as pl
from jax.experimental.pallas import tpu as pltpu
from jax.experimental.pallas import tpu_sc as plsc

EMB = 128
HID = 512
DIN = 256
N_OBJ = 10000
N_TRI = 160000

_BF = jnp.bfloat16
_F32 = jnp.float32

_CHUNK = 128
_NCHUNK = N_TRI // _CHUNK
_NCHUNK_PAD = 1280
_N_TRI_PAD = _NCHUNK_PAD * _CHUNK
_NW = 32
_GITER = _NCHUNK_PAD // _NW
_SITER = _NCHUNK_PAD // 16
_NCHUNK1 = N_TRI // _CHUNK
_CITER = (_NCHUNK1 + 15) // 16
_N_OBJ_PAD = 10240
_ROWS_PER_SUB = _N_OBJ_PAD // 16
_DUMMY_ROW = N_OBJ
_NPARTS = 4


def _dotbf(a, b):
    return jnp.dot(a.astype(_BF), b.astype(_BF), preferred_element_type=_F32)


def _dothi(a, b):
    return jax.lax.dot_general(
        a, b, (((1,), (0,)), ((), ())),
        precision=jax.lax.Precision.HIGHEST, preferred_element_type=_F32)


def _sc_mesh():
    return plsc.VectorSubcoreMesh(core_axis_name="c", subcore_axis_name="s")


def _pack2(hi, lo):
    hb = jax.lax.bitcast_convert_type(hi.astype(_BF).astype(_F32), jnp.int32)
    lb = jax.lax.bitcast_convert_type(lo.astype(_BF).astype(_F32), jnp.int32)
    return (hb & jnp.int32(-65536)) | jax.lax.shift_right_logical(lb, 16)


def _unpack2(p):
    hi = jax.lax.bitcast_convert_type(p & jnp.int32(-65536), _F32)
    lo = jax.lax.bitcast_convert_type(p << 16, _F32)
    return hi, lo


def _sc_gather(table, sidx2, oidx2):
    d = table.shape[1]
    nchunk = sidx2.shape[0]
    giter = nchunk // _NW
    assert giter % 8 in (0, 2, 4)
    fetchn = giter + {0: 0, 2: 6, 4: 4}[giter % 8]
    out_sds = jax.ShapeDtypeStruct((nchunk * _CHUNK, d), table.dtype)

    @pl.kernel(
        out_type=(out_sds, out_sds),
        mesh=_sc_mesh(),
        scratch_types=[
            pltpu.VMEM((fetchn, _CHUNK), jnp.int32),
            pltpu.VMEM((_CHUNK, d), table.dtype),
            pltpu.VMEM((_CHUNK, d), table.dtype),
            pltpu.SemaphoreType.DMA,
            pltpu.SemaphoreType.DMA,
        ],
    )
    def k(table_hbm, sidx_hbm, oidx_hbm, sg_hbm, og_hbm,
          idx_v, rows0, rows1, sem0, sem1):
        ci = lax.axis_index("c")
        si = lax.axis_index("s")
        w = si * 2 + ci
        c0 = w * giter
        mod = c0 % 8
        a0 = pl.multiple_of(c0 - mod, 8)

        def do_side(idx_hbm, out_hbm):
            pltpu.sync_copy(idx_hbm.at[pl.ds(a0, fetchn)], idx_v)

            def start(j, buf, sem):
                pltpu.async_copy(table_hbm.at[idx_v.at[mod + j]], buf, sem)

            def wait(buf, sem):
                pltpu.make_async_copy(table_hbm.at[idx_v.at[0]], buf,
                                      sem).wait()

            start(0, rows0, sem0)

            @pl.loop(0, giter // 2)
            def _(t):
                j0 = 2 * t
                j1 = j0 + 1
                wait(rows0, sem0)
                start(j1, rows1, sem1)
                pltpu.sync_copy(rows0,
                                out_hbm.at[pl.ds((c0 + j0) * _CHUNK, _CHUNK)])

                @pl.when(t < giter // 2 - 1)
                def _():
                    start(j0 + 2, rows0, sem0)

                wait(rows1, sem1)
                pltpu.sync_copy(rows1,
                                out_hbm.at[pl.ds((c0 + j1) * _CHUNK, _CHUNK)])

        do_side(sidx_hbm, sg_hbm)
        do_side(oidx_hbm, og_hbm)

    return k(table, sidx2, oidx2)


def _sc_scatter(new_s, new_o, sidx2, oidx2, zrows):
    nchunk = sidx2.shape[0]
    siter = nchunk // 16
    assert siter % 8 in (0, 4)
    fetchs = siter + 4 if siter % 8 == 4 else siter
    out_sds = jax.ShapeDtypeStruct((_N_OBJ_PAD, HID), _F32)

    @pl.kernel(
        out_type=out_sds,
        mesh=_sc_mesh(),
        scratch_types=[
            pltpu.VMEM_SHARED((_N_OBJ_PAD, _CHUNK), _F32),
            pltpu.VMEM((fetchs, _CHUNK), jnp.int32),
            pltpu.VMEM((_CHUNK, _CHUNK), _F32),
            pltpu.VMEM((_CHUNK, _CHUNK), _F32),
            pltpu.SemaphoreType.DMA,
            pltpu.SemaphoreType.DMA,
        ],
    )
    def k(ns_hbm, no_hbm, sidx_hbm, oidx_hbm, z_hbm, out_hbm,
          shared, idx_v, rows0, rows1, sem0, sem1):
        ci = lax.axis_index("c")
        si = lax.axis_index("s")
        row0_ = si * _ROWS_PER_SUB
        c0 = si * siter
        smod = c0 % 8
        sa0 = pl.multiple_of(c0 - smod, 8)

        for p in range(2):
            col0 = (ci * 2 + p) * _CHUNK
            pltpu.sync_copy(z_hbm, shared.at[pl.ds(row0_, _ROWS_PER_SUB)])
            plsc.subcore_barrier()

            def do_side(val_hbm, idx_hbm, col0=col0):
                pltpu.sync_copy(idx_hbm.at[pl.ds(sa0, fetchs)], idx_v)

                def start(j, buf, sem):
                    pltpu.async_copy(
                        val_hbm.at[pl.ds((c0 + j) * _CHUNK, _CHUNK),
                                   pl.ds(col0, _CHUNK)],
                        buf, sem)

                def wait(buf, sem):
                    pltpu.make_async_copy(
                        val_hbm.at[pl.ds(0, _CHUNK), pl.ds(0, _CHUNK)],
                        buf, sem).wait()

                start(0, rows0, sem0)

                @pl.loop(0, siter // 2)
                def _(t):
                    j0 = 2 * t
                    j1 = j0 + 1
                    wait(rows0, sem0)
                    start(j1, rows1, sem1)
                    pltpu.sync_copy(rows0, shared.at[idx_v.at[smod + j0]],
                                    add=True)

                    @pl.when(t < siter // 2 - 1)
                    def _():
                        start(j0 + 2, rows0, sem0)

                    wait(rows1, sem1)
                    pltpu.sync_copy(rows1, shared.at[idx_v.at[smod + j1]],
                                    add=True)

            do_side(ns_hbm, sidx_hbm)
            do_side(no_hbm, oidx_hbm)
            plsc.subcore_barrier()
            pltpu.sync_copy(
                shared.at[pl.ds(row0_, _ROWS_PER_SUB)],
                out_hbm.at[pl.ds(row0_, _ROWS_PER_SUB), pl.ds(col0, _CHUNK)])
            plsc.subcore_barrier()

    return k(new_s, new_o, sidx2, oidx2, zrows)


def _sc_counts(s_idx, o_idx, ones128, zrows):
    out_sds = jax.ShapeDtypeStruct((_N_OBJ_PAD, _CHUNK), _F32)

    @pl.kernel(
        out_type=out_sds,
        mesh=_sc_mesh(),
        scratch_types=[
            pltpu.VMEM_SHARED((_N_OBJ_PAD, _CHUNK), _F32),
            pltpu.VMEM((_CHUNK,), jnp.int32),
            pltpu.VMEM((_CHUNK, _CHUNK), _F32),
        ],
    )
    def k(sidx_hbm, oidx_hbm, ones_hbm, z_hbm, cnt_hbm, shared, idx_v, ones_v):
        ci = lax.axis_index("c")
        si = lax.axis_index("s")
        row0 = si * _ROWS_PER_SUB
        pltpu.sync_copy(z_hbm, shared.at[pl.ds(row0, _ROWS_PER_SUB)])
        pltpu.sync_copy(ones_hbm, ones_v)
        plsc.subcore_barrier()

        for idx_hbm in (sidx_hbm, oidx_hbm):
            @pl.loop(0, _SITER)
            def _(j, idx_hbm=idx_hbm):
                c = si + j * 16

                @pl.when(c < _NCHUNK)
                def _():
                    pltpu.sync_copy(idx_hbm.at[pl.ds(c * _CHUNK, _CHUNK)], idx_v)
                    pltpu.sync_copy(ones_v, shared.at[idx_v], add=True)

        plsc.subcore_barrier()
        half = _N_OBJ_PAD // 2
        wrow = ci * half + si * (half // 16)
        pltpu.sync_copy(shared.at[pl.ds(wrow, half // 16)],
                        cnt_hbm.at[pl.ds(wrow, half // 16)])
        plsc.subcore_barrier()

    return k(s_idx, o_idx, ones128, zrows)


_BN = 1000
_BE = 640
_NEB = N_TRI // _BE


def _full(shape):
    nd = len(shape)
    return pl.BlockSpec(shape, lambda i: (0,) * nd)


def _tc_embed(objs3, boxes_p, emb_p, d3w_p, d3b):
    def body(objs_ref, bx_ref, emb_ref, d3w_ref, d3b_ref, o_ref):
        obj = objs_ref[0, 0, :]
        oh = (obj[:, None] == lax.broadcasted_iota(jnp.int32, (_BN, 128), 1))
        sel = _dothi(oh.astype(_F32), emb_ref[...])
        d3 = _dotbf(bx_ref[...], d3w_ref[...]) + d3b_ref[...]
        o_ref[...] = _pack2(sel, d3)

    return pl.pallas_call(
        body,
        grid=(N_OBJ // _BN,),
        in_specs=[
            pl.BlockSpec((1, 1, _BN), lambda i: (i, 0, 0)),
            pl.BlockSpec((_BN, 128), lambda i: (i, 0)),
            _full((128, EMB)),
            _full((128, EMB)),
            _full((1, EMB)),
        ],
        out_specs=pl.BlockSpec((_BN, EMB), lambda i: (i, 0)),
        out_shape=jax.ShapeDtypeStruct((N_OBJ, EMB), jnp.int32),
    )(objs3, boxes_p, emb_p, d3w_p, d3b)


def _tc_edge(sg, og, pstream, p3, pred_emb_p, lp, w1ap_next):
    w1a = lp['W1a']
    w1a_s = w1a[:DIN]
    w1a_p = w1a[DIN:2 * DIN]
    w1a_o = w1a[2 * DIN:]
    b1a = lp['b1a'].reshape(1, HID)
    b1b = lp['b1b'].reshape(1, 2 * HID + DIN)

    first = pstream is None
    last = w1ap_next is None

    def body(*refs):
        i = 0
        sg_ref = refs[i]; i += 1
        og_ref = refs[i]; i += 1
        if first:
            p3_ref = refs[i]; i += 1
            pemb_ref = refs[i]; i += 1
            w1ap0_ref = refs[i]; i += 1
        else:
            ps_ref = refs[i]; i += 1
        w1as_ref = refs[i]; i += 1
        w1ao_ref = refs[i]; i += 1
        b1a_ref = refs[i]; i += 1
        w1b_ref = refs[i]; i += 1
        b1b_ref = refs[i]; i += 1
        if not last:
            w1apn_ref = refs[i]; i += 1
        ns_ref = refs[i]; i += 1
        no_ref = refs[i]; i += 1
        if not last:
            pn_ref = refs[i]; i += 1

        if first:
            pe1 = _dotbf(pemb_ref[...], w1ap0_ref[...])
            pid = p3_ref[0, 0, :]
            oh = (pid[:, None] ==
                  lax.broadcasted_iota(jnp.int32, (_BE, 128), 1))
            pterm = _dothi(oh.astype(_F32), pe1)
        else:
            pterm = ps_ref[...]

        s_hi, s_lo = _unpack2(sg_ref[...])
        o_hi, o_lo = _unpack2(og_ref[...])
        h = _dotbf(s_hi, w1as_ref[:EMB]) + _dotbf(s_lo, w1as_ref[EMB:])
        h = h + _dotbf(o_hi, w1ao_ref[:EMB]) + _dotbf(o_lo, w1ao_ref[EMB:])
        h = jnp.maximum(h + pterm + b1a_ref[...], 0.0)
        t = _dotbf(h, w1b_ref[...]) + b1b_ref[...]
        ns_ref[...] = jnp.maximum(t[:, :HID], 0.0)
        no_ref[...] = jnp.maximum(t[:, HID + DIN:], 0.0)
        if not last:
            newp = jnp.maximum(t[:, HID:HID + DIN], 0.0)
            pn_ref[...] = _dotbf(newp, w1apn_ref[...])

    in_specs = [
        pl.BlockSpec((_BE, EMB), lambda i: (i, 0)),
        pl.BlockSpec((_BE, EMB), lambda i: (i, 0)),
    ]
    args = [sg, og]
    if first:
        in_specs += [pl.BlockSpec((1, 1, _BE), lambda i: (i, 0, 0)),
                     _full((128, DIN)), _full((DIN, HID))]
        args += [p3, pred_emb_p, w1a_p]
    else:
        in_specs += [pl.BlockSpec((_BE, HID), lambda i: (i, 0))]
        args += [pstream]
    in_specs += [_full((DIN, HID)), _full((DIN, HID)), _full((1, HID)),
                 _full((HID, 2 * HID + DIN)), _full((1, 2 * HID + DIN))]
    args += [w1a_s, w1a_o, b1a, lp['W1b'], b1b]
    if not last:
        in_specs += [_full((DIN, HID))]
        args += [w1ap_next]

    nrows = sg.shape[0]
    out_specs = [pl.BlockSpec((_BE, HID), lambda i: (i, 0)),
                 pl.BlockSpec((_BE, HID), lambda i: (i, 0))]
    out_shape = [jax.ShapeDtypeStruct((nrows, HID), _F32),
                 jax.ShapeDtypeStruct((nrows, HID), _F32)]
    if not last:
        out_specs += [pl.BlockSpec((_BE, HID), lambda i: (i, 0))]
        out_shape += [jax.ShapeDtypeStruct((nrows, HID), _F32)]

    return pl.pallas_call(
        body, grid=(sg.shape[0] // _BE,), in_specs=in_specs,
        out_specs=out_specs, out_shape=out_shape,
    )(*args)


def _tc_node(pooled_list, cnts, lp):
    b2a = lp['b2a'].reshape(1, HID)
    b2b = lp['b2b'].reshape(1, DIN)
    n = len(pooled_list)

    def body(*refs):
        p_refs = refs[:n]
        cnt_ref, w2a_ref, b2a_ref, w2b_ref, b2b_ref, o_ref = refs[n:]
        cnt = cnt_ref[:, 0:1]
        pool = p_refs[0][...]
        for pr in p_refs[1:]:
            pool = pool + pr[...]
        pool = pool / jnp.maximum(cnt, 1.0)
        h = jnp.maximum(_dotbf(pool, w2a_ref[...]) + b2a_ref[...], 0.0)
        obj = jnp.maximum(_dotbf(h, w2b_ref[...]) + b2b_ref[...], 0.0)
        o_ref[...] = _pack2(obj[:, :EMB], obj[:, EMB:])

    return pl.pallas_call(
        body,
        grid=(N_OBJ // _BN,),
        in_specs=(
            [pl.BlockSpec((_BN, HID), lambda i: (i, 0))] * n + [
             pl.BlockSpec((_BN, _CHUNK), lambda i: (i, 0)),
             _full((HID, HID)), _full((1, HID)),
             _full((HID, DIN)), _full((1, DIN))]),
        out_specs=pl.BlockSpec((_BN, EMB), lambda i: (i, 0)),
        out_shape=jax.ShapeDtypeStruct((N_OBJ, EMB), jnp.int32),
    )(*pooled_list, cnts, lp['W2a'], b2a, lp['W2b'], b2b)


def _tc_head(obj, params):
    mvb1 = params['mv_b1'].reshape(1, HID)
    mvb2 = params['mv_b2'].reshape(1, DIN)
    meanb = params['mean_b'].reshape(1, EMB)
    varb = params['var_b'].reshape(1, EMB)

    def body(o_ref, w1_ref, b1_ref, w2_ref, b2_ref,
             mw_ref, mb_ref, vw_ref, vb_ref, mu_ref, lv_ref):
        o_hi, o_lo = _unpack2(o_ref[...])
        h = jnp.maximum(_dotbf(o_hi, w1_ref[:EMB]) +
                        _dotbf(o_lo, w1_ref[EMB:]) + b1_ref[...], 0.0)
        h = jnp.maximum(_dotbf(h, w2_ref[...]) + b2_ref[...], 0.0)
        mu_ref[...] = _dotbf(h, mw_ref[...]) + mb_ref[...]
        lv_ref[...] = _dotbf(h, vw_ref[...]) + vb_ref[...]

    return pl.pallas_call(
        body,
        grid=(N_OBJ // _BN,),
        in_specs=[
            pl.BlockSpec((_BN, EMB), lambda i: (i, 0)),
            _full((DIN, HID)), _full((1, HID)),
            _full((HID, DIN)), _full((1, DIN)),
            _full((DIN, EMB)), _full((1, EMB)),
            _full((DIN, EMB)), _full((1, EMB)),
        ],
        out_specs=[pl.BlockSpec((_BN, EMB), lambda i: (i, 0)),
                   pl.BlockSpec((_BN, EMB), lambda i: (i, 0))],
        out_shape=[jax.ShapeDtypeStruct((N_OBJ, EMB), _F32),
                   jax.ShapeDtypeStruct((N_OBJ, EMB), _F32)],
    )(obj, params['mv_W1'], mvb1, params['mv_W2'], mvb2,
      params['mean_W'], meanb, params['var_W'], varb)


def kernel(objs, triples, boxes_gt, attributes, enc_text_feat, enc_rel_feat,
           params):
    objs = objs.astype(jnp.int32)
    triples = triples.astype(jnp.int32)
    s_idx = triples[:, 0]
    p_idx = triples[:, 1]
    o_idx = triples[:, 2]

    objs3 = objs.reshape(N_OBJ // _BN, 1, _BN)
    boxes_p = jnp.pad(boxes_gt, ((0, 0), (0, 128 - 6)))
    emb_p = jnp.pad(params['obj_emb_ec'], ((0, 128 - (params['obj_emb_ec'].shape[0])), (0, 0)))
    d3w_p = jnp.pad(params['d3_W'], ((0, 128 - 6), (0, 0)))
    d3b = params['d3_b'].reshape(1, EMB)
    pred_emb_p = jnp.pad(params['pred_emb_ec'], ((0, 128 - 16), (0, 0)))
    zrows = jnp.zeros((_ROWS_PER_SUB, _CHUNK), _F32)
    ones128 = jnp.ones((_CHUNK, _CHUNK), _F32)
    npad = _N_TRI_PAD - N_TRI
    nparts = _NPARTS
    part = _NCHUNK_PAD // nparts

    def _parts(x):
        return tuple(x[i * part:(i + 1) * part] for i in range(nparts))

    sidx_g2 = _parts(jnp.pad(s_idx, (0, npad)).reshape(_NCHUNK_PAD, _CHUNK))
    oidx_g2 = _parts(jnp.pad(o_idx, (0, npad)).reshape(_NCHUNK_PAD, _CHUNK))
    sidx_s2 = _parts(jnp.pad(s_idx, (0, npad), constant_values=_DUMMY_ROW)
                     .reshape(_NCHUNK_PAD, _CHUNK))
    oidx_s2 = _parts(jnp.pad(o_idx, (0, npad), constant_values=_DUMMY_ROW)
                     .reshape(_NCHUNK_PAD, _CHUNK))
    p3p = jnp.pad(p_idx, (0, npad)).reshape(_N_TRI_PAD // _BE, 1, _BE)
    nb_part = _N_TRI_PAD // _BE // nparts
    p3h = tuple(p3p[i * nb_part:(i + 1) * nb_part] for i in range(nparts))

    gconv = params['gconv']

    obj_vecs = _tc_embed(objs3, boxes_p, emb_p, d3w_p, d3b)
    cnts = _sc_counts(s_idx, o_idx, ones128, zrows)

    pstream = [None] * nparts
    for l in range(3):
        lp = gconv[l]
        w1ap_next = (gconv[l + 1]['W1a'][DIN:2 * DIN] if l < 2 else None)
        pooled = []
        for h in range(nparts):
            sg, og = _sc_gather(obj_vecs, sidx_g2[h], oidx_g2[h])
            outs = _tc_edge(sg, og, pstream[h], p3h[h] if l == 0 else None,
                            pred_emb_p if l == 0 else None, lp, w1ap_next)
            if l < 2:
                new_s, new_o, pstream[h] = outs
            else:
                new_s, new_o = outs
            pooled.append(
                _sc_scatter(new_s, new_o, sidx_s2[h], oidx_s2[h], zrows))
        obj_vecs = _tc_node(pooled, cnts, lp)

    mu, logvar = _tc_head(obj_vecs, params)
    return (mu, logvar)


# speedup vs baseline: 1.0770x; 1.0770x over previous
import jax
import jax.numpy as jnp
from jax import lax
from jax.experimental import pallas as pl
from jax.experimental.pallas import tpu as pltpu
from jax.experimental.pallas import tpu_sc as plsc

EMB = 128
HID = 512
DIN = 256
N_OBJ = 10000
N_TRI = 160000

_BF = jnp.bfloat16
_F32 = jnp.float32

_CHUNK = 128
_NCHUNK = N_TRI // _CHUNK
_NCHUNK_PAD = 1280
_N_TRI_PAD = _NCHUNK_PAD * _CHUNK
_NW = 32
_GITER = _NCHUNK_PAD // _NW
_SITER = _NCHUNK_PAD // 16
_NCHUNK1 = N_TRI // _CHUNK
_CITER = (_NCHUNK1 + 15) // 16
_N_OBJ_PAD = 10240
_ROWS_PER_SUB = _N_OBJ_PAD // 16
_DUMMY_ROW = N_OBJ
_NPARTS = 2


def _dotbf(a, b):
    return jnp.dot(a.astype(_BF), b.astype(_BF), preferred_element_type=_F32)


def _dothi(a, b):
    return jax.lax.dot_general(
        a, b, (((1,), (0,)), ((), ())),
        precision=jax.lax.Precision.HIGHEST, preferred_element_type=_F32)


def _sc_mesh():
    return plsc.VectorSubcoreMesh(core_axis_name="c", subcore_axis_name="s")


def _pack2(hi, lo):
    hb = jax.lax.bitcast_convert_type(hi.astype(_BF).astype(_F32), jnp.int32)
    lb = jax.lax.bitcast_convert_type(lo.astype(_BF).astype(_F32), jnp.int32)
    return (hb & jnp.int32(-65536)) | jax.lax.shift_right_logical(lb, 16)


def _unpack2(p):
    hi = jax.lax.bitcast_convert_type(p & jnp.int32(-65536), _F32)
    lo = jax.lax.bitcast_convert_type(p << 16, _F32)
    return hi, lo


def _sc_gather(table, sidx2, oidx2):
    d = table.shape[1]
    nchunk = sidx2.shape[0]
    giter = nchunk // _NW
    assert giter % 8 in (0, 2, 4)
    fetchn = giter + {0: 0, 2: 6, 4: 4}[giter % 8]
    out_sds = jax.ShapeDtypeStruct((nchunk * _CHUNK, d), table.dtype)

    @pl.kernel(
        out_type=(out_sds, out_sds),
        mesh=_sc_mesh(),
        scratch_types=[
            pltpu.VMEM((fetchn, _CHUNK), jnp.int32),
            pltpu.VMEM((_CHUNK, d), table.dtype),
            pltpu.VMEM((_CHUNK, d), table.dtype),
            pltpu.SemaphoreType.DMA,
            pltpu.SemaphoreType.DMA,
        ],
    )
    def k(table_hbm, sidx_hbm, oidx_hbm, sg_hbm, og_hbm,
          idx_v, rows0, rows1, sem0, sem1):
        ci = lax.axis_index("c")
        si = lax.axis_index("s")
        w = si * 2 + ci
        c0 = w * giter
        mod = c0 % 8
        a0 = pl.multiple_of(c0 - mod, 8)

        def do_side(idx_hbm, out_hbm):
            pltpu.sync_copy(idx_hbm.at[pl.ds(a0, fetchn)], idx_v)

            def start(j, buf, sem):
                pltpu.async_copy(table_hbm.at[idx_v.at[mod + j]], buf, sem)

            def wait(buf, sem):
                pltpu.make_async_copy(table_hbm.at[idx_v.at[0]], buf,
                                      sem).wait()

            start(0, rows0, sem0)

            @pl.loop(0, giter // 2)
            def _(t):
                j0 = 2 * t
                j1 = j0 + 1
                wait(rows0, sem0)
                start(j1, rows1, sem1)
                pltpu.sync_copy(rows0,
                                out_hbm.at[pl.ds((c0 + j0) * _CHUNK, _CHUNK)])

                @pl.when(t < giter // 2 - 1)
                def _():
                    start(j0 + 2, rows0, sem0)

                wait(rows1, sem1)
                pltpu.sync_copy(rows1,
                                out_hbm.at[pl.ds((c0 + j1) * _CHUNK, _CHUNK)])

        do_side(sidx_hbm, sg_hbm)
        do_side(oidx_hbm, og_hbm)

    return k(table, sidx2, oidx2)


def _sc_scatter(new_s, new_o, sidx2, oidx2, zrows):
    nchunk = sidx2.shape[0]
    siter = nchunk // 16
    assert siter % 8 in (0, 4)
    fetchs = siter + 4 if siter % 8 == 4 else siter
    out_sds = jax.ShapeDtypeStruct((_N_OBJ_PAD, HID), _F32)

    @pl.kernel(
        out_type=out_sds,
        mesh=_sc_mesh(),
        scratch_types=[
            pltpu.VMEM_SHARED((_N_OBJ_PAD, _CHUNK), _F32),
            pltpu.VMEM((fetchs, _CHUNK), jnp.int32),
            pltpu.VMEM((_CHUNK, _CHUNK), _F32),
            pltpu.VMEM((_CHUNK, _CHUNK), _F32),
            pltpu.SemaphoreType.DMA,
            pltpu.SemaphoreType.DMA,
        ],
    )
    def k(ns_hbm, no_hbm, sidx_hbm, oidx_hbm, z_hbm, out_hbm,
          shared, idx_v, rows0, rows1, sem0, sem1):
        ci = lax.axis_index("c")
        si = lax.axis_index("s")
        row0_ = si * _ROWS_PER_SUB
        c0 = si * siter
        smod = c0 % 8
        sa0 = pl.multiple_of(c0 - smod, 8)

        for p in range(2):
            col0 = (ci * 2 + p) * _CHUNK
            pltpu.sync_copy(z_hbm, shared.at[pl.ds(row0_, _ROWS_PER_SUB)])
            plsc.subcore_barrier()

            def do_side(val_hbm, idx_hbm, col0=col0):
                pltpu.sync_copy(idx_hbm.at[pl.ds(sa0, fetchs)], idx_v)

                def start(j, buf, sem):
                    pltpu.async_copy(
                        val_hbm.at[pl.ds((c0 + j) * _CHUNK, _CHUNK),
                                   pl.ds(col0, _CHUNK)],
                        buf, sem)

                def wait(buf, sem):
                    pltpu.make_async_copy(
                        val_hbm.at[pl.ds(0, _CHUNK), pl.ds(0, _CHUNK)],
                        buf, sem).wait()

                start(0, rows0, sem0)

                @pl.loop(0, siter // 2)
                def _(t):
                    j0 = 2 * t
                    j1 = j0 + 1
                    wait(rows0, sem0)
                    start(j1, rows1, sem1)
                    pltpu.sync_copy(rows0, shared.at[idx_v.at[smod + j0]],
                                    add=True)

                    @pl.when(t < siter // 2 - 1)
                    def _():
                        start(j0 + 2, rows0, sem0)

                    wait(rows1, sem1)
                    pltpu.sync_copy(rows1, shared.at[idx_v.at[smod + j1]],
                                    add=True)

            do_side(ns_hbm, sidx_hbm)
            do_side(no_hbm, oidx_hbm)
            plsc.subcore_barrier()
            pltpu.sync_copy(
                shared.at[pl.ds(row0_, _ROWS_PER_SUB)],
                out_hbm.at[pl.ds(row0_, _ROWS_PER_SUB), pl.ds(col0, _CHUNK)])
            plsc.subcore_barrier()

    return k(new_s, new_o, sidx2, oidx2, zrows)


def _sc_counts(s_idx, o_idx, ones128, zrows):
    out_sds = jax.ShapeDtypeStruct((_N_OBJ_PAD, _CHUNK), _F32)

    @pl.kernel(
        out_type=out_sds,
        mesh=_sc_mesh(),
        scratch_types=[
            pltpu.VMEM_SHARED((_N_OBJ_PAD, _CHUNK), _F32),
            pltpu.VMEM((_CHUNK,), jnp.int32),
            pltpu.VMEM((_CHUNK, _CHUNK), _F32),
        ],
    )
    def k(sidx_hbm, oidx_hbm, ones_hbm, z_hbm, cnt_hbm, shared, idx_v, ones_v):
        ci = lax.axis_index("c")
        si = lax.axis_index("s")
        row0 = si * _ROWS_PER_SUB
        pltpu.sync_copy(z_hbm, shared.at[pl.ds(row0, _ROWS_PER_SUB)])
        pltpu.sync_copy(ones_hbm, ones_v)
        plsc.subcore_barrier()

        for idx_hbm in (sidx_hbm, oidx_hbm):
            @pl.loop(0, _SITER)
            def _(j, idx_hbm=idx_hbm):
                c = si + j * 16

                @pl.when(c < _NCHUNK)
                def _():
                    pltpu.sync_copy(idx_hbm.at[pl.ds(c * _CHUNK, _CHUNK)], idx_v)
                    pltpu.sync_copy(ones_v, shared.at[idx_v], add=True)

        plsc.subcore_barrier()
        half = _N_OBJ_PAD // 2
        wrow = ci * half + si * (half // 16)
        pltpu.sync_copy(shared.at[pl.ds(wrow, half // 16)],
                        cnt_hbm.at[pl.ds(wrow, half // 16)])
        plsc.subcore_barrier()

    return k(s_idx, o_idx, ones128, zrows)


_BN = 1000
_BE = 640
_NEB = N_TRI // _BE


def _full(shape):
    nd = len(shape)
    return pl.BlockSpec(shape, lambda i: (0,) * nd)


def _tc_embed(objs3, boxes_p, emb_p, d3w_p, d3b):
    def body(objs_ref, bx_ref, emb_ref, d3w_ref, d3b_ref, o_ref):
        obj = objs_ref[0, 0, :]
        oh = (obj[:, None] == lax.broadcasted_iota(jnp.int32, (_BN, 128), 1))
        sel = _dothi(oh.astype(_F32), emb_ref[...])
        d3 = _dotbf(bx_ref[...], d3w_ref[...]) + d3b_ref[...]
        o_ref[...] = _pack2(sel, d3)

    return pl.pallas_call(
        body,
        grid=(N_OBJ // _BN,),
        in_specs=[
            pl.BlockSpec((1, 1, _BN), lambda i: (i, 0, 0)),
            pl.BlockSpec((_BN, 128), lambda i: (i, 0)),
            _full((128, EMB)),
            _full((128, EMB)),
            _full((1, EMB)),
        ],
        out_specs=pl.BlockSpec((_BN, EMB), lambda i: (i, 0)),
        out_shape=jax.ShapeDtypeStruct((N_OBJ, EMB), jnp.int32),
    )(objs3, boxes_p, emb_p, d3w_p, d3b)


def _tc_edge(sg, og, pstream, p3, pred_emb_p, lp, w1ap_next):
    w1a = lp['W1a']
    w1a_s = w1a[:DIN]
    w1a_p = w1a[DIN:2 * DIN]
    w1a_o = w1a[2 * DIN:]
    b1a = lp['b1a'].reshape(1, HID)
    b1b = lp['b1b'].reshape(1, 2 * HID + DIN)

    first = pstream is None
    last = w1ap_next is None

    def body(*refs):
        i = 0
        sg_ref = refs[i]; i += 1
        og_ref = refs[i]; i += 1
        if first:
            p3_ref = refs[i]; i += 1
            pemb_ref = refs[i]; i += 1
            w1ap0_ref = refs[i]; i += 1
        else:
            ps_ref = refs[i]; i += 1
        w1as_ref = refs[i]; i += 1
        w1ao_ref = refs[i]; i += 1
        b1a_ref = refs[i]; i += 1
        w1b_ref = refs[i]; i += 1
        b1b_ref = refs[i]; i += 1
        if not last:
            w1apn_ref = refs[i]; i += 1
        ns_ref = refs[i]; i += 1
        no_ref = refs[i]; i += 1
        if not last:
            pn_ref = refs[i]; i += 1

        if first:
            pe1 = _dotbf(pemb_ref[...], w1ap0_ref[...])
            pid = p3_ref[0, 0, :]
            oh = (pid[:, None] ==
                  lax.broadcasted_iota(jnp.int32, (_BE, 128), 1))
            pterm = _dothi(oh.astype(_F32), pe1)
        else:
            pterm = ps_ref[...]

        s_hi, s_lo = _unpack2(sg_ref[...])
        o_hi, o_lo = _unpack2(og_ref[...])
        h = _dotbf(s_hi, w1as_ref[:EMB]) + _dotbf(s_lo, w1as_ref[EMB:])
        h = h + _dotbf(o_hi, w1ao_ref[:EMB]) + _dotbf(o_lo, w1ao_ref[EMB:])
        h = jnp.maximum(h + pterm + b1a_ref[...], 0.0)
        t = _dotbf(h, w1b_ref[...]) + b1b_ref[...]
        ns_ref[...] = jnp.maximum(t[:, :HID], 0.0)
        no_ref[...] = jnp.maximum(t[:, HID + DIN:], 0.0)
        if not last:
            newp = jnp.maximum(t[:, HID:HID + DIN], 0.0)
            pn_ref[...] = _dotbf(newp, w1apn_ref[...])

    in_specs = [
        pl.BlockSpec((_BE, EMB), lambda i: (i, 0)),
        pl.BlockSpec((_BE, EMB), lambda i: (i, 0)),
    ]
    args = [sg, og]
    if first:
        in_specs += [pl.BlockSpec((1, 1, _BE), lambda i: (i, 0, 0)),
                     _full((128, DIN)), _full((DIN, HID))]
        args += [p3, pred_emb_p, w1a_p]
    else:
        in_specs += [pl.BlockSpec((_BE, HID), lambda i: (i, 0))]
        args += [pstream]
    in_specs += [_full((DIN, HID)), _full((DIN, HID)), _full((1, HID)),
                 _full((HID, 2 * HID + DIN)), _full((1, 2 * HID + DIN))]
    args += [w1a_s, w1a_o, b1a, lp['W1b'], b1b]
    if not last:
        in_specs += [_full((DIN, HID))]
        args += [w1ap_next]

    nrows = sg.shape[0]
    out_specs = [pl.BlockSpec((_BE, HID), lambda i: (i, 0)),
                 pl.BlockSpec((_BE, HID), lambda i: (i, 0))]
    out_shape = [jax.ShapeDtypeStruct((nrows, HID), _F32),
                 jax.ShapeDtypeStruct((nrows, HID), _F32)]
    if not last:
        out_specs += [pl.BlockSpec((_BE, HID), lambda i: (i, 0))]
        out_shape += [jax.ShapeDtypeStruct((nrows, HID), _F32)]

    return pl.pallas_call(
        body, grid=(sg.shape[0] // _BE,), in_specs=in_specs,
        out_specs=out_specs, out_shape=out_shape,
    )(*args)


def _tc_node(pooled_list, cnts, lp):
    b2a = lp['b2a'].reshape(1, HID)
    b2b = lp['b2b'].reshape(1, DIN)
    n = len(pooled_list)

    def body(*refs):
        p_refs = refs[:n]
        cnt_ref, w2a_ref, b2a_ref, w2b_ref, b2b_ref, o_ref = refs[n:]
        cnt = cnt_ref[:, 0:1]
        pool = p_refs[0][...]
        for pr in p_refs[1:]:
            pool = pool + pr[...]
        pool = pool / jnp.maximum(cnt, 1.0)
        h = jnp.maximum(_dotbf(pool, w2a_ref[...]) + b2a_ref[...], 0.0)
        obj = jnp.maximum(_dotbf(h, w2b_ref[...]) + b2b_ref[...], 0.0)
        o_ref[...] = _pack2(obj[:, :EMB], obj[:, EMB:])

    return pl.pallas_call(
        body,
        grid=(N_OBJ // _BN,),
        in_specs=(
            [pl.BlockSpec((_BN, HID), lambda i: (i, 0))] * n + [
             pl.BlockSpec((_BN, _CHUNK), lambda i: (i, 0)),
             _full((HID, HID)), _full((1, HID)),
             _full((HID, DIN)), _full((1, DIN))]),
        out_specs=pl.BlockSpec((_BN, EMB), lambda i: (i, 0)),
        out_shape=jax.ShapeDtypeStruct((N_OBJ, EMB), jnp.int32),
    )(*pooled_list, cnts, lp['W2a'], b2a, lp['W2b'], b2b)


def _tc_head(obj, params):
    mvb1 = params['mv_b1'].reshape(1, HID)
    mvb2 = params['mv_b2'].reshape(1, DIN)
    meanb = params['mean_b'].reshape(1, EMB)
    varb = params['var_b'].reshape(1, EMB)

    def body(o_ref, w1_ref, b1_ref, w2_ref, b2_ref,
             mw_ref, mb_ref, vw_ref, vb_ref, mu_ref, lv_ref):
        o_hi, o_lo = _unpack2(o_ref[...])
        h = jnp.maximum(_dotbf(o_hi, w1_ref[:EMB]) +
                        _dotbf(o_lo, w1_ref[EMB:]) + b1_ref[...], 0.0)
        h = jnp.maximum(_dotbf(h, w2_ref[...]) + b2_ref[...], 0.0)
        mu_ref[...] = _dotbf(h, mw_ref[...]) + mb_ref[...]
        lv_ref[...] = _dotbf(h, vw_ref[...]) + vb_ref[...]

    return pl.pallas_call(
        body,
        grid=(N_OBJ // _BN,),
        in_specs=[
            pl.BlockSpec((_BN, EMB), lambda i: (i, 0)),
            _full((DIN, HID)), _full((1, HID)),
            _full((HID, DIN)), _full((1, DIN)),
            _full((DIN, EMB)), _full((1, EMB)),
            _full((DIN, EMB)), _full((1, EMB)),
        ],
        out_specs=[pl.BlockSpec((_BN, EMB), lambda i: (i, 0)),
                   pl.BlockSpec((_BN, EMB), lambda i: (i, 0))],
        out_shape=[jax.ShapeDtypeStruct((N_OBJ, EMB), _F32),
                   jax.ShapeDtypeStruct((N_OBJ, EMB), _F32)],
    )(obj, params['mv_W1'], mvb1, params['mv_W2'], mvb2,
      params['mean_W'], meanb, params['var_W'], varb)


def kernel(objs, triples, boxes_gt, attributes, enc_text_feat, enc_rel_feat,
           params):
    objs = objs.astype(jnp.int32)
    triples = triples.astype(jnp.int32)
    s_idx = triples[:, 0]
    p_idx = triples[:, 1]
    o_idx = triples[:, 2]

    objs3 = objs.reshape(N_OBJ // _BN, 1, _BN)
    boxes_p = jnp.pad(boxes_gt, ((0, 0), (0, 128 - 6)))
    emb_p = jnp.pad(params['obj_emb_ec'], ((0, 128 - (params['obj_emb_ec'].shape[0])), (0, 0)))
    d3w_p = jnp.pad(params['d3_W'], ((0, 128 - 6), (0, 0)))
    d3b = params['d3_b'].reshape(1, EMB)
    pred_emb_p = jnp.pad(params['pred_emb_ec'], ((0, 128 - 16), (0, 0)))
    zrows = jnp.zeros((_ROWS_PER_SUB, _CHUNK), _F32)
    ones128 = jnp.ones((_CHUNK, _CHUNK), _F32)
    npad = _N_TRI_PAD - N_TRI
    nparts = _NPARTS
    part = _NCHUNK_PAD // nparts

    def _parts(x):
        return tuple(x[i * part:(i + 1) * part] for i in range(nparts))

    sidx_g2 = _parts(jnp.pad(s_idx, (0, npad)).reshape(_NCHUNK_PAD, _CHUNK))
    oidx_g2 = _parts(jnp.pad(o_idx, (0, npad)).reshape(_NCHUNK_PAD, _CHUNK))
    sidx_s2 = _parts(jnp.pad(s_idx, (0, npad), constant_values=_DUMMY_ROW)
                     .reshape(_NCHUNK_PAD, _CHUNK))
    oidx_s2 = _parts(jnp.pad(o_idx, (0, npad), constant_values=_DUMMY_ROW)
                     .reshape(_NCHUNK_PAD, _CHUNK))
    p3p = jnp.pad(p_idx, (0, npad)).reshape(_N_TRI_PAD // _BE, 1, _BE)
    nb_part = _N_TRI_PAD // _BE // nparts
    p3h = tuple(p3p[i * nb_part:(i + 1) * nb_part] for i in range(nparts))

    gconv = params['gconv']

    obj_vecs = _tc_embed(objs3, boxes_p, emb_p, d3w_p, d3b)
    cnts = _sc_counts(s_idx, o_idx, ones128, zrows)

    pstream = [None] * nparts
    for l in range(3):
        lp = gconv[l]
        w1ap_next = (gconv[l + 1]['W1a'][DIN:2 * DIN] if l < 2 else None)
        pooled = []
        for h in range(nparts):
            sg, og = _sc_gather(obj_vecs, sidx_g2[h], oidx_g2[h])
            outs = _tc_edge(sg, og, pstream[h], p3h[h] if l == 0 else None,
                            pred_emb_p if l == 0 else None, lp, w1ap_next)
            if l < 2:
                new_s, new_o, pstream[h] = outs
            else:
                new_s, new_o = outs
            pooled.append(
                _sc_scatter(new_s, new_o, sidx_s2[h], oidx_s2[h], zrows))
        obj_vecs = _tc_node(pooled, cnts, lp)

    mu, logvar = _tc_head(obj_vecs, params)
    return (mu, logvar)
